# baseline (speedup 1.0000x reference)
import functools

import jax
import jax.numpy as jnp
from jax import lax
from jax.experimental import pallas as pl
from jax.experimental.pallas import tpu as pltpu
from jax.experimental.pallas import tpu_sc as plsc

NB_USER = 1000000
NB_ITEM = 1000000
F = 32
B = 16384
RW = 128

NC, NS, L = 2, 16, 16
NW = NC * NS
BPW = B // NW
HALF = BPW // 2

RANGE = 31232
CW = 1024
NFULL = 30
TAILW = NB_USER - NW * RANGE
WL = 1024
SLOTS = 48
NSG = SLOTS // L
IB = 2048
SENT = 1 << 30


def _gather_body(users_hbm, items_hbm, uet_hbm, iet_hbm,
                 urows_hbm, irows_hbm,
                 blk, wr, wb, cwr, cwb, pan, pC, pD, stg, six2,
                 sp, sq, ss):
    wid = lax.axis_index("s") * NC + lax.axis_index("c")
    lo = wid * RANGE
    hi = lo + RANGE + jnp.where(wid == NW - 1, TAILW, 0)
    dummy = B + wid
    lane = lax.iota(jnp.int32, L)

    for tbl_hbm, idx_hbm, rows_hbm in ((uet_hbm, users_hbm, urows_hbm),
                                       (iet_hbm, items_hbm, irows_hbm)):
        def pwait(tbl_hbm=tbl_hbm):
            pltpu.make_async_copy(tbl_hbm.at[:, pl.ds(0, CW)],
                                  pan.at[:, pl.ds(0, CW)], sp).wait()

        def swait(rows_hbm=rows_hbm):
            pltpu.make_async_copy(stg.at[pl.ds(0, SLOTS)],
                                  rows_hbm.at[pl.ds(0, SLOTS)], ss).wait()

        pltpu.async_copy(tbl_hbm.at[:, pl.ds(lo, CW)],
                         pan.at[:, pl.ds(0, CW)], sp)
        c30 = pltpu.async_copy(
            tbl_hbm.at[:, pl.ds(lo + NFULL * CW, 512)], pC, sq)
        c31 = pltpu.async_copy(
            tbl_hbm.at[:, pl.ds(NW * RANGE, TAILW)], pD, sq)

        cnt = jnp.int32(0)
        for s in range(B // IB):
            pltpu.sync_copy(idx_hbm.at[pl.ds(s * IB, IB)], blk)

            def scan_g(g, cnt, s=s):
                v = blk[pl.ds(g * L, L)]
                m = (v >= lo) & (v < hi)
                plsc.store_compressed(wr.at[pl.ds(cnt, L)], v - lo, mask=m)
                bv = lane + (s * IB + g * L)
                plsc.store_compressed(wb.at[pl.ds(cnt, L)], bv, mask=m)
                pc = plsc.all_reduce_population_count(m)[0]
                return jnp.minimum(cnt + pc, WL)

            cnt = lax.fori_loop(0, IB // L, scan_g, cnt)
        wr[pl.ds(cnt, L)] = jnp.full((L,), SENT, jnp.int32)
        ngrp = (cnt + L - 1) // L

        def select(base, w, coff, sel, panel_ref, rows_hbm=rows_hbm):
            soff = sel * SLOTS

            def rescan(j, cs):
                v = wr[pl.ds(j * L, L)]
                pb = wb[pl.ds(j * L, L)]
                m = (v >= base) & (v < base + w)
                plsc.store_compressed(cwr.at[pl.ds(cs, L)], v - base, mask=m)
                plsc.store_compressed(cwb.at[pl.ds(cs, L)], pb, mask=m)
                pc = plsc.all_reduce_population_count(m)[0]
                return jnp.minimum(cs + pc, SLOTS)

            cslot = lax.fori_loop(0, ngrp, rescan, jnp.int32(0))

            def ext(hg, carry):
                cols = cwr[pl.ds(hg * L, L)]
                cols = jnp.minimum(jnp.maximum(cols, 0), w - 1) + coff
                hv = lane + hg * L + soff
                for f in range(F):
                    fv = jnp.full((L,), f, jnp.int32)
                    v = plsc.load_gather(panel_ref, [fv, cols])
                    plsc.store_scatter(stg, [hv, fv], v)
                return carry

            _ = ext
            selv = jnp.full((L,), 0, jnp.int32) + sel
            for t in range(NSG):
                live = (lane + t * L) < cslot
                sixv = jnp.where(live, cwb[pl.ds(t * L, L)], dummy)
                plsc.store_scatter(six2, [selv, lane + t * L], sixv)
            pltpu.async_copy(
                stg.at[pl.ds(pl.multiple_of(soff, L), SLOTS)],
                rows_hbm.at[six2.at[sel]], ss)

        def chunk(k, carry, tbl_hbm=tbl_hbm, rows_hbm=rows_hbm):
            sel = k % 2

            @pl.when(k + 1 < NFULL)
            def _():
                off = pl.multiple_of((k + 1) * CW, CW)
                doff = pl.multiple_of(((k + 1) % 2) * CW, CW)
                pltpu.async_copy(tbl_hbm.at[:, pl.ds(lo + off, CW)],
                                 pan.at[:, pl.ds(doff, CW)], sp)

            pwait()

            @pl.when(k >= 2)
            def _():
                swait()

            select(k * CW, CW, sel * CW, sel, pan)
            return carry

        lax.fori_loop(0, NFULL, chunk, 0)

        c30.wait()
        swait()
        select(NFULL * CW, 512, 0, jnp.int32(0), pC)
        c31.wait()
        swait()
        select(RANGE, TAILW, 0, jnp.int32(1), pD)
        swait()
        swait()


def _dot_body(users_hbm, items_hbm, urows_hbm, irows_hbm, ub_hbm, ib_hbm,
              gb_hbm, out_hbm,
              uidx, iidx, ur, ir, ubias, ibias, gbv, ob, sr, sb, sg):
    wid = lax.axis_index("s") * NC + lax.axis_index("c")
    base = wid * BPW
    lane = lax.iota(jnp.int32, L)

    pltpu.sync_copy(users_hbm.at[pl.ds(base, BPW)], uidx)
    pltpu.sync_copy(items_hbm.at[pl.ds(base, BPW)], iidx)
    cub = pltpu.async_copy(ub_hbm.at[uidx], ubias, sb)
    cib = pltpu.async_copy(ib_hbm.at[iidx], ibias, sb)
    cgb = pltpu.async_copy(gb_hbm, gbv.at[pl.ds(0, 1)], sg)

    for half in range(2):
        cu = pltpu.async_copy(
            urows_hbm.at[pl.ds(base + half * HALF, HALF)], ur, sr)
        ci = pltpu.async_copy(
            irows_hbm.at[pl.ds(base + half * HALF, HALF)], ir, sr)
        cu.wait()
        ci.wait()

        def group(g, carry, half=half):
            rows = lane + g * L
            acc = jnp.zeros((L,), jnp.float32)
            for f in range(F):
                fv = jnp.full((L,), f, jnp.int32)
                acc = acc + (plsc.load_gather(ur, [rows, fv])
                             * plsc.load_gather(ir, [rows, fv]))
            ob[pl.ds(half * HALF + g * L, L)] = acc
            return carry

        lax.fori_loop(0, HALF // L, group, 0)

    cub.wait()
    cib.wait()
    cgb.wait()
    gb = gbv[...][0]

    def biasadd(g, carry):
        s = pl.ds(g * L, L)
        ob[s] = ob[s] + ubias[s] + ibias[s] + gb
        return carry

    lax.fori_loop(0, BPW // L, biasadd, 0)
    pltpu.sync_copy(ob, out_hbm.at[pl.ds(base, BPW)])


@jax.jit
def _mf(users, items, user_embeddings, item_embeddings, user_biases,
        item_biases, global_bias):
    mesh = plsc.VectorSubcoreMesh(core_axis_name="c", subcore_axis_name="s")
    cp = pltpu.CompilerParams(needs_layout_passes=False,
                              use_tc_tiling_on_sc=True)
    gather = pl.kernel(
        _gather_body,
        out_type=(jax.ShapeDtypeStruct((B + NW, RW), jnp.float32),
                  jax.ShapeDtypeStruct((B + NW, RW), jnp.float32)),
        mesh=mesh,
        compiler_params=cp,
        scratch_types=[
            pltpu.VMEM((IB,), jnp.int32),
            pltpu.VMEM((WL + L,), jnp.int32),
            pltpu.VMEM((WL + L,), jnp.int32),
            pltpu.VMEM((SLOTS + L,), jnp.int32),
            pltpu.VMEM((SLOTS + L,), jnp.int32),
            pltpu.VMEM((F, 2 * CW), jnp.float32),
            pltpu.VMEM((F, 512), jnp.float32),
            pltpu.VMEM((F, TAILW), jnp.float32),
            pltpu.VMEM((2 * SLOTS, RW), jnp.float32),
            pltpu.VMEM((2, SLOTS), jnp.int32),
            pltpu.SemaphoreType.DMA,
            pltpu.SemaphoreType.DMA,
            pltpu.SemaphoreType.DMA,
        ],
    )
    dot = pl.kernel(
        _dot_body,
        out_type=jax.ShapeDtypeStruct((B,), jnp.float32),
        mesh=mesh,
        compiler_params=cp,
        scratch_types=[
            pltpu.VMEM((BPW,), jnp.int32),
            pltpu.VMEM((BPW,), jnp.int32),
            pltpu.VMEM((HALF, RW), jnp.float32),
            pltpu.VMEM((HALF, RW), jnp.float32),
            pltpu.VMEM((BPW,), jnp.float32),
            pltpu.VMEM((BPW,), jnp.float32),
            pltpu.VMEM((L,), jnp.float32),
            pltpu.VMEM((BPW,), jnp.float32),
            pltpu.SemaphoreType.DMA,
            pltpu.SemaphoreType.DMA,
            pltpu.SemaphoreType.DMA,
        ],
    )
    users = users.astype(jnp.int32)
    items = items.astype(jnp.int32)
    urows, irows = gather(users, items, user_embeddings.T, item_embeddings.T)
    out = dot(users, items, urows, irows,
              user_biases.reshape(NB_USER), item_biases.reshape(NB_ITEM),
              global_bias)
    return out.reshape(B, 1)


def kernel(users, items, user_embeddings, item_embeddings, user_biases,
           item_biases, global_bias):
    return _mf(users, items, user_embeddings, item_embeddings, user_biases,
               item_biases, global_bias)


# speedup vs baseline: 1.7336x; 1.7336x over previous
import functools

import jax
import jax.numpy as jnp
from jax import lax
from jax.experimental import pallas as pl
from jax.experimental.pallas import tpu as pltpu
from jax.experimental.pallas import tpu_sc as plsc

NB_USER = 1000000
NB_ITEM = 1000000
F = 32
B = 16384
RW = 128

NC, NS, L = 2, 16, 16
NW = NC * NS
BPW = B // NW
HALF = BPW // 2

RANGE = 31232
CW = 1024
NFULL = 30
TAILW = NB_USER - NW * RANGE
WL = 1024
SLOTS = 48
NSG = SLOTS // L
IB = 2048
NPAD = 2048
SENT = 1 << 30


def _gather_body(users_hbm, items_hbm, uet_hbm, iet_hbm,
                 urows_hbm, irows_hbm,
                 blk, wr, wb, cwr, cwb, pan, pC, pD, stg, six2,
                 sp, sq, ss):
    wid = lax.axis_index("s") * NC + lax.axis_index("c")
    lo = wid * RANGE
    hi = lo + RANGE + jnp.where(wid == NW - 1, TAILW, 0)
    lane = lax.iota(jnp.int32, L)

    for tbl_hbm, idx_hbm, rows_hbm in ((uet_hbm, users_hbm, urows_hbm),
                                       (iet_hbm, items_hbm, irows_hbm)):
        def pwait(tbl_hbm=tbl_hbm):
            pltpu.make_async_copy(tbl_hbm.at[:, pl.ds(0, CW)],
                                  pan.at[:, pl.ds(0, CW)], sp).wait()

        def swait(rows_hbm=rows_hbm):
            pltpu.make_async_copy(stg.at[pl.ds(0, SLOTS)],
                                  rows_hbm.at[pl.ds(0, SLOTS)], ss).wait()

        pltpu.async_copy(tbl_hbm.at[:, pl.ds(lo, CW)],
                         pan.at[:, pl.ds(0, CW)], sp)
        c30 = pltpu.async_copy(
            tbl_hbm.at[:, pl.ds(lo + NFULL * CW, 512)], pC, sq)
        c31 = pltpu.async_copy(
            tbl_hbm.at[:, pl.ds(NW * RANGE, TAILW)], pD, sq)

        cnt = jnp.int32(0)
        for s in range(B // IB):
            pltpu.sync_copy(idx_hbm.at[pl.ds(s * IB, IB)], blk)

            def scan_g(g, cnt, s=s):
                v = blk[pl.ds(g * L, L)]
                m = (v >= lo) & (v < hi)
                plsc.store_compressed(wr.at[pl.ds(cnt, L)], v - lo, mask=m)
                bv = lane + (s * IB + g * L)
                plsc.store_compressed(wb.at[pl.ds(cnt, L)], bv, mask=m)
                pc = plsc.all_reduce_population_count(m)[0]
                return jnp.minimum(cnt + pc, WL)

            cnt = lax.fori_loop(0, IB // L, scan_g, cnt)
        wr[pl.ds(cnt, L)] = jnp.full((L,), SENT, jnp.int32)
        ngrp = (cnt + L - 1) // L

        def select(base, w, coff, sel, panel_ref, rows_hbm=rows_hbm):
            soff = sel * SLOTS

            def rescan(j, cs):
                v = wr[pl.ds(j * L, L)]
                pb = wb[pl.ds(j * L, L)]
                m = (v >= base) & (v < base + w)
                plsc.store_compressed(cwr.at[pl.ds(cs, L)], v - base, mask=m)
                plsc.store_compressed(cwb.at[pl.ds(cs, L)], pb, mask=m)
                pc = plsc.all_reduce_population_count(m)[0]
                return jnp.minimum(cs + pc, SLOTS)

            cslot = lax.fori_loop(0, ngrp, rescan, jnp.int32(0))

            def ext(hg, carry):
                cols = cwr[pl.ds(hg * L, L)]
                cols = jnp.minimum(jnp.maximum(cols, 0), w - 1) + coff
                hv = lane + hg * L + soff
                for f in range(F):
                    fv = jnp.full((L,), f, jnp.int32)
                    v = plsc.load_gather(panel_ref, [fv, cols])
                    plsc.store_scatter(stg, [hv, fv], v)
                return carry

            lax.fori_loop(0, (cslot + L - 1) // L, ext, 0)
            selv = jnp.full((L,), 0, jnp.int32) + sel
            for t in range(NSG):
                live = (lane + t * L) < cslot
                pad = B + ((wid * SLOTS + lane + t * L) & (NPAD - 1))
                sixv = jnp.where(live, cwb[pl.ds(t * L, L)], pad)
                plsc.store_scatter(six2, [selv, lane + t * L], sixv)
            pltpu.async_copy(
                stg.at[pl.ds(pl.multiple_of(soff, L), SLOTS)],
                rows_hbm.at[six2.at[sel]], ss)

        def chunk(k, carry, tbl_hbm=tbl_hbm, rows_hbm=rows_hbm):
            sel = k % 2

            @pl.when(k + 1 < NFULL)
            def _():
                off = pl.multiple_of((k + 1) * CW, CW)
                doff = pl.multiple_of(((k + 1) % 2) * CW, CW)
                pltpu.async_copy(tbl_hbm.at[:, pl.ds(lo + off, CW)],
                                 pan.at[:, pl.ds(doff, CW)], sp)

            pwait()

            @pl.when(k >= 2)
            def _():
                swait()

            select(k * CW, CW, sel * CW, sel, pan)
            return carry

        lax.fori_loop(0, NFULL, chunk, 0)

        c30.wait()
        swait()
        select(NFULL * CW, 512, 0, jnp.int32(0), pC)
        c31.wait()
        swait()
        select(RANGE, TAILW, 0, jnp.int32(1), pD)
        swait()
        swait()


def _dot_body(users_hbm, items_hbm, urows_hbm, irows_hbm, ub_hbm, ib_hbm,
              gb_hbm, out_hbm,
              uidx, iidx, ur, ir, ubias, ibias, gbv, ob, sr, sb, sg):
    wid = lax.axis_index("s") * NC + lax.axis_index("c")
    base = wid * BPW
    lane = lax.iota(jnp.int32, L)

    pltpu.sync_copy(users_hbm.at[pl.ds(base, BPW)], uidx)
    pltpu.sync_copy(items_hbm.at[pl.ds(base, BPW)], iidx)
    cub = pltpu.async_copy(ub_hbm.at[uidx], ubias, sb)
    cib = pltpu.async_copy(ib_hbm.at[iidx], ibias, sb)
    cgb = pltpu.async_copy(gb_hbm, gbv.at[pl.ds(0, 1)], sg)

    for half in range(2):
        cu = pltpu.async_copy(
            urows_hbm.at[pl.ds(base + half * HALF, HALF)], ur, sr)
        ci = pltpu.async_copy(
            irows_hbm.at[pl.ds(base + half * HALF, HALF)], ir, sr)
        cu.wait()
        ci.wait()

        def group(g, carry, half=half):
            rows = lane + g * L
            acc = jnp.zeros((L,), jnp.float32)
            for f in range(F):
                fv = jnp.full((L,), f, jnp.int32)
                acc = acc + (plsc.load_gather(ur, [rows, fv])
                             * plsc.load_gather(ir, [rows, fv]))
            ob[pl.ds(half * HALF + g * L, L)] = acc
            return carry

        lax.fori_loop(0, HALF // L, group, 0)

    cub.wait()
    cib.wait()
    cgb.wait()
    gb = gbv[...][0]

    def biasadd(g, carry):
        s = pl.ds(g * L, L)
        ob[s] = ob[s] + ubias[s] + ibias[s] + gb
        return carry

    lax.fori_loop(0, BPW // L, biasadd, 0)
    pltpu.sync_copy(ob, out_hbm.at[pl.ds(base, BPW)])


@jax.jit
def _mf(users, items, user_embeddings, item_embeddings, user_biases,
        item_biases, global_bias):
    mesh = plsc.VectorSubcoreMesh(core_axis_name="c", subcore_axis_name="s")
    cp = pltpu.CompilerParams(needs_layout_passes=False,
                              use_tc_tiling_on_sc=True)
    gather = pl.kernel(
        _gather_body,
        out_type=(jax.ShapeDtypeStruct((B + NPAD, RW), jnp.float32),
                  jax.ShapeDtypeStruct((B + NPAD, RW), jnp.float32)),
        mesh=mesh,
        compiler_params=cp,
        scratch_types=[
            pltpu.VMEM((IB,), jnp.int32),
            pltpu.VMEM((WL + L,), jnp.int32),
            pltpu.VMEM((WL + L,), jnp.int32),
            pltpu.VMEM((SLOTS + L,), jnp.int32),
            pltpu.VMEM((SLOTS + L,), jnp.int32),
            pltpu.VMEM((F, 2 * CW), jnp.float32),
            pltpu.VMEM((F, 512), jnp.float32),
            pltpu.VMEM((F, TAILW), jnp.float32),
            pltpu.VMEM((2 * SLOTS, RW), jnp.float32),
            pltpu.VMEM((2, SLOTS), jnp.int32),
            pltpu.SemaphoreType.DMA,
            pltpu.SemaphoreType.DMA,
            pltpu.SemaphoreType.DMA,
        ],
    )
    dot = pl.kernel(
        _dot_body,
        out_type=jax.ShapeDtypeStruct((B,), jnp.float32),
        mesh=mesh,
        compiler_params=cp,
        scratch_types=[
            pltpu.VMEM((BPW,), jnp.int32),
            pltpu.VMEM((BPW,), jnp.int32),
            pltpu.VMEM((HALF, RW), jnp.float32),
            pltpu.VMEM((HALF, RW), jnp.float32),
            pltpu.VMEM((BPW,), jnp.float32),
            pltpu.VMEM((BPW,), jnp.float32),
            pltpu.VMEM((L,), jnp.float32),
            pltpu.VMEM((BPW,), jnp.float32),
            pltpu.SemaphoreType.DMA,
            pltpu.SemaphoreType.DMA,
            pltpu.SemaphoreType.DMA,
        ],
    )
    users = users.astype(jnp.int32)
    items = items.astype(jnp.int32)
    urows, irows = gather(users, items, user_embeddings.T, item_embeddings.T)
    out = dot(users, items, urows, irows,
              user_biases.reshape(NB_USER), item_biases.reshape(NB_ITEM),
              global_bias)
    return out.reshape(B, 1)


def kernel(users, items, user_embeddings, item_embeddings, user_biases,
           item_biases, global_bias):
    return _mf(users, items, user_embeddings, item_embeddings, user_biases,
               item_biases, global_bias)
